# baseline (speedup 1.0000x reference)
import functools

import jax
import jax.numpy as jnp
from jax import lax
from jax.experimental import pallas as pl
from jax.experimental.pallas import tpu as pltpu
from jax.experimental.pallas import tpu_sc as plsc

B = 64
L = 448
D = 1024
P = 7
N = B * L

NC, NS = 2, 16
BG = 4
LG = 8
BPG = B // BG
LPG = L // LG
SUB = 8
NSUB = LPG // SUB
STEPS = NSUB * BPG
P8 = 8
PB = LPG * P8
JT = 4
NJT = D // (JT * 16)

_GTR_DNUMS = lax.GatherDimensionNumbers(
    offset_dims=(), collapsed_slice_dims=(0,), start_index_map=(0,))

_MESH = plsc.VectorSubcoreMesh(
    core_axis_name="c", subcore_axis_name="s", num_cores=NC, num_subcores=NS
)


@functools.partial(
    pl.kernel,
    out_type=jax.ShapeDtypeStruct((N, D), jnp.float32),
    mesh=_MESH,
    scratch_types=[
        pltpu.VMEM((BPG * LPG,), jnp.int32),
        pltpu.VMEM((BPG * PB,), jnp.float32),
        pltpu.VMEM((P * D,), jnp.float32),
        pltpu.VMEM((SUB, D), jnp.float32),
        pltpu.VMEM((2, SUB, D), jnp.float32),
        pltpu.SemaphoreType.DMA,
        pltpu.SemaphoreType.DMA,
        pltpu.SemaphoreType.DMA,
    ],
)
def _sc_fused(table, ids, pros, w, pos2, out,
              idx_v, pros_v, w_v, pos_loc, gbuf, sem_in, sem_g, sem_o):
    wid = lax.axis_index("s") * NC + lax.axis_index("c")
    lg = wid // BG
    bg = wid % BG
    b0 = bg * BPG
    l0 = lg * LPG

    cps = []
    for i in range(BPG):
        row = pl.multiple_of((b0 + i) * L + l0, 8)
        cps.append(pltpu.async_copy(
            ids.at[pl.ds(row, LPG)], idx_v.at[pl.ds(i * LPG, LPG)], sem_in))
        cps.append(pltpu.async_copy(
            pros.at[pl.ds(pl.multiple_of(row * P8, 8), PB)],
            pros_v.at[pl.ds(i * PB, PB)], sem_in))
    cps.append(pltpu.async_copy(w, w_v, sem_in))
    for cp in cps:
        cp.wait()

    def issue_gather(k, par):
        sc_i = k // BPG
        b_i = k % BPG
        off = pl.multiple_of(b_i * LPG + sc_i * SUB, 8)
        return pltpu.async_copy(
            table.at[idx_v.at[pl.ds(off, SUB)]], gbuf.at[par], sem_g)

    def wait_gather(par):
        pltpu.make_async_copy(
            table.at[idx_v.at[pl.ds(0, SUB)]], gbuf.at[par], sem_g).wait()

    def wait_out(par):
        pltpu.make_async_copy(gbuf.at[par], out.at[pl.ds(0, SUB)], sem_o).wait()

    def compute(k, par):
        sc_i = k // BPG
        b_i = k % BPG
        pbase = b_i * PB + sc_i * SUB * P8

        def jt_body(jt, carry):
            woff = jt * (JT * 16)
            wv = [[w_v[pl.ds(pp * D + woff + jj * 16, 16)] for jj in range(JT)]
                  for pp in range(P)]
            for t in range(SUB):
                if t % 2 == 0:
                    pv = pros_v[pl.ds(
                        pl.multiple_of(pbase + (t // 2) * 16, 16), 16)]
                bp = [lax.gather(
                          pv,
                          jnp.full((16, 1), (t % 2) * P8 + pp, jnp.int32),
                          _GTR_DNUMS, (1,),
                          mode=lax.GatherScatterMode.PROMISE_IN_BOUNDS)
                      for pp in range(P)]
                for jj in range(JT):
                    sl = pl.ds(woff + jj * 16, 16)
                    acc = gbuf[par, t, sl] + pos_loc[t, sl]
                    for pp in range(P):
                        acc = acc + bp[pp] * wv[pp][jj]
                    gbuf[par, t, sl] = acc
            return carry

        lax.fori_loop(0, NJT, jt_body, 0)

    issue_gather(0, 0)

    def loop_body(i, carry):
        for par in (0, 1):
            k = 2 * i + par
            sc_i = k // BPG
            b_i = k % BPG
            if par == 0:
                @pl.when(b_i == 0)
                def _():
                    pltpu.sync_copy(
                        pos2.at[pl.ds(l0 + sc_i * SUB, SUB)], pos_loc)

            @pl.when(k >= 1)
            def _():
                wait_out(1 - par)

            @pl.when(k < STEPS - 1)
            def _():
                issue_gather(k + 1, 1 - par)

            wait_gather(par)
            compute(k, par)
            out_row = (b0 + b_i) * L + l0 + sc_i * SUB
            pltpu.async_copy(gbuf.at[par], out.at[pl.ds(out_row, SUB)], sem_o)
        return carry

    lax.fori_loop(0, STEPS // 2, loop_body, 0)
    wait_out(1)


def _pos2_body(pos_ref, b_ref, o_ref):
    o_ref[...] = pos_ref[...] + b_ref[...]


def kernel(token_ids, prosody_features, token_table, pos_table, proj_w, proj_b):
    ids = token_ids.reshape(N).astype(jnp.int32)
    pros = jnp.pad(prosody_features.reshape(N, P),
                   ((0, 0), (0, P8 - P))).reshape(N * P8)
    w = proj_w.reshape(P * D)
    pos2 = pl.pallas_call(
        _pos2_body,
        out_shape=jax.ShapeDtypeStruct((L, D), jnp.float32),
    )(pos_table, proj_b.reshape(1, D))
    out = _sc_fused(token_table, ids, pros, w, pos2)
    return out.reshape(B, L, D)


# speedup vs baseline: 4.2457x; 4.2457x over previous
import functools

import jax
import jax.numpy as jnp
from jax import lax
from jax.experimental import pallas as pl
from jax.experimental.pallas import tpu as pltpu
from jax.experimental.pallas import tpu_sc as plsc

B = 64
L = 448
D = 1024
P = 7
N = B * L

G = 4
BC = B // G
NG = N // G

NC, NS = 2, 16
NW = NC * NS
BPW = NG // NW
SUB = 56
NSTEP = BPW // SUB

_MESH = plsc.VectorSubcoreMesh(
    core_axis_name="c", subcore_axis_name="s", num_cores=NC, num_subcores=NS
)


@functools.partial(
    pl.kernel,
    out_type=jax.ShapeDtypeStruct((NG, D), jnp.float32),
    mesh=_MESH,
    scratch_types=[
        pltpu.VMEM((BPW,), jnp.int32),
        pltpu.VMEM((2, SUB, D), jnp.float32),
        pltpu.SemaphoreType.DMA,
        pltpu.SemaphoreType.DMA,
    ],
)
def _sc_gather(table, ids, out, idx_v, buf, sem_g, sem_o):
    wid = lax.axis_index("s") * NC + lax.axis_index("c")
    base = wid * BPW
    pltpu.sync_copy(ids.at[pl.ds(pl.multiple_of(base, 8), BPW)], idx_v)

    def issue_gather(c, par):
        return pltpu.async_copy(
            table.at[idx_v.at[pl.ds(c * SUB, SUB)]], buf.at[par], sem_g)

    issue_gather(0, 0)
    for c in range(NSTEP):
        par = c % 2
        if c >= 1:
            pltpu.make_async_copy(
                buf.at[1 - par], out.at[pl.ds(0, SUB)], sem_o).wait()
        if c < NSTEP - 1:
            issue_gather(c + 1, 1 - par)
        pltpu.make_async_copy(
            table.at[idx_v.at[pl.ds(0, SUB)]], buf.at[par], sem_g).wait()
        pltpu.async_copy(
            buf.at[par], out.at[pl.ds(base + c * SUB, SUB)], sem_o)
    pltpu.make_async_copy(
        buf.at[0 if NSTEP % 2 else 1], out.at[pl.ds(0, SUB)], sem_o).wait()


def _tc_fuse_body(*refs):
    tok_ref, pos_ref, pros_ref, w_ref, b_ref = refs[:5]
    out_ref = refs[-1]
    proj = jax.lax.dot_general(
        pros_ref[...], w_ref[...],
        dimension_numbers=(((1,), (0,)), ((), ())),
        preferred_element_type=jnp.float32,
    )
    out_ref[...] = tok_ref[...] + pos_ref[...] + proj + b_ref[...]


def _tc_fuse(g, tok_g, pos_table, pros_g, proj_w, proj_b2, prev_out):
    in_specs = [
        pl.BlockSpec((L, D), lambda i: (i, 0)),
        pl.BlockSpec((L, D), lambda i: (0, 0)),
        pl.BlockSpec((L, P), lambda i: (i, 0)),
        pl.BlockSpec((P, D), lambda i: (0, 0)),
        pl.BlockSpec((1, D), lambda i: (0, 0)),
    ]
    args = [tok_g, pos_table, pros_g, proj_w, proj_b2]
    aliases = {}
    if prev_out is not None:
        in_specs.append(pl.BlockSpec(memory_space=pl.ANY))
        args.append(prev_out)
        aliases = {5: 0}
    return pl.pallas_call(
        _tc_fuse_body,
        grid=(BC,),
        in_specs=in_specs,
        out_specs=pl.BlockSpec((L, D), lambda i, g=g: (g * BC + i, 0)),
        out_shape=jax.ShapeDtypeStruct((N, D), jnp.float32),
        input_output_aliases=aliases,
    )(*args)


def kernel(token_ids, prosody_features, token_table, pos_table, proj_w, proj_b):
    ids = token_ids.reshape(N).astype(jnp.int32)
    pros = prosody_features.reshape(N, P)
    proj_b2 = proj_b.reshape(1, D)
    toks = [_sc_gather(token_table, lax.dynamic_slice_in_dim(ids, g * NG, NG))
            for g in range(G)]
    out = None
    for g in range(G):
        pros_g = lax.dynamic_slice_in_dim(pros, g * NG, NG)
        out = _tc_fuse(g, toks[g], pos_table, pros_g, proj_w, proj_b2, out)
    return out.reshape(B, L, D)
